# Initial kernel scaffold; baseline (speedup 1.0000x reference)
import jax
import jax.numpy as jnp
from jax.experimental import pallas as pl


def kernel(x, weights, edge_index, edge_weights, edge_attr, chunk_eye, W_gcn, W_gin1, W_gin2, W_gat, W_sage_self, W_sage_nbr, W_graph1, W_graph2, W_arma1, W_arma2, W_mlp1, W_mlp2, be_gcn, be_gin, be_gat, be_sage, be_graph, be_arma, a_src, a_dst, eps_gin, ln_gamma, ln_beta):
    raise NotImplementedError("write your pallas kernel here")


# speedup vs baseline: 2.1309x; 2.1309x over previous
import functools

import jax
import jax.numpy as jnp
from jax.experimental import pallas as pl

D = 256
NUM_OPS = 7
CS = 32

_DEST = {g: [] for g in range(8)}
for _o in range(7):
    for _b in range(8):
        _idx0 = _o * 256 + _b * 32
        _DEST[_idx0 // 224].append((_o, _b, (_idx0 % 224) // 32))


def _dense_body(x_ref, s1_ref, s2_ref, s3_ref, s4_ref, s5_ref, c_ref,
                invc_ref, wx_ref, ws1_ref, wgin1_ref, wsnbr_ref, wgr2_ref,
                wgin2_ref, wmlp2_ref, pc_ref, lng_ref, lnb_ref, wmix_ref,
                out_ref):
    f32 = jnp.float32
    x = x_ref[...]
    xw = jnp.dot(x, wx_ref[...], preferred_element_type=f32)
    s1w = jnp.dot(s1_ref[...], ws1_ref[...], preferred_element_type=f32)
    cp = jnp.dot(c_ref[...], pc_ref[...], preferred_element_type=f32)
    gin_pre = xw[:, 0:256] + jnp.dot(s2_ref[...], wgin1_ref[...],
                                     preferred_element_type=f32)
    gin = jnp.dot(jax.nn.relu(gin_pre), wgin2_ref[...],
                  preferred_element_type=f32)
    gcn = s1w[:, 0:256] + cp[:, 0:256]
    gat = s3_ref[...] + cp[:, 256:512]
    invc = invc_ref[...]
    sage = xw[:, 256:512] + invc * (
        jnp.dot(s4_ref[...], wsnbr_ref[...], preferred_element_type=f32)
        + cp[:, 512:768])
    graph = (xw[:, 512:768]
             + jnp.dot(s5_ref[...], wgr2_ref[...], preferred_element_type=f32)
             + cp[:, 768:1024])
    arma = jax.nn.relu(s1w[:, 256:512] + cp[:, 1024:1280] + xw[:, 768:1024])
    mlp = jnp.dot(jax.nn.relu(xw[:, 1024:1280]), wmlp2_ref[...],
                  preferred_element_type=f32)

    gamma = lng_ref[...]
    beta = lnb_ref[...]
    zs = []
    for op in (gcn, gin, gat, sage, graph, arma, mlp):
        mu = jnp.mean(op, axis=1, keepdims=True)
        ctr = op - mu
        var = jnp.mean(ctr * ctr, axis=1, keepdims=True)
        zs.append(ctr * jax.lax.rsqrt(var + 1e-5) * gamma + beta)

    wmix = wmix_ref[...]
    cols = []
    for g in range(8):
        acc = None
        for (o, b, _j) in _DEST[g]:
            term = zs[o][:, b * CS:(b + 1) * CS] * wmix[o:o + 1, b * CS:(b + 1) * CS]
            acc = term if acc is None else acc + term
        cols.append(acc)
    out_ref[...] = jnp.concatenate(cols, axis=1)


def _dense_stage(x, s1, s2, s3, s4, s5, ccat, invc, wx, ws1, wgin1, wsnbr,
                 wgr2, wgin2, wmlp2, pc, lng, lnb, wmix):
    n = x.shape[0]
    blk = 1000
    grid = (n // blk,)
    big = lambda: pl.BlockSpec((blk, D), lambda i: (i, 0))
    full = lambda a: pl.BlockSpec(a.shape, lambda i: tuple(0 for _ in a.shape))
    return pl.pallas_call(
        _dense_body,
        grid=grid,
        in_specs=[
            big(), big(), big(), big(), big(), big(),
            pl.BlockSpec((blk, 64), lambda i: (i, 0)),
            pl.BlockSpec((blk, 1), lambda i: (i, 0)),
            full(wx), full(ws1), full(wgin1), full(wsnbr), full(wgr2),
            full(wgin2), full(wmlp2), full(pc), full(lng), full(lnb),
            full(wmix),
        ],
        out_specs=big(),
        out_shape=jax.ShapeDtypeStruct((n, D), jnp.float32),
    )(x, s1, s2, s3, s4, s5, ccat, invc, wx, ws1, wgin1, wsnbr, wgr2, wgin2,
      wmlp2, pc, lng, lnb, wmix)


def kernel(x, weights, edge_index, edge_weights, edge_attr, chunk_eye, W_gcn,
           W_gin1, W_gin2, W_gat, W_sage_self, W_sage_nbr, W_graph1, W_graph2,
           W_arma1, W_arma2, W_mlp1, W_mlp2, be_gcn, be_gin, be_gat, be_sage,
           be_graph, be_arma, a_src, a_dst, eps_gin, ln_gamma, ln_beta):
    n = x.shape[0]
    src, dst = edge_index[0], edge_index[1]
    ew = edge_weights
    off = jnp.array([0, 5, 10], dtype=edge_attr.dtype)
    eidx = edge_attr + off[None, :]

    def seg(d, i):
        return jax.ops.segment_sum(d, i, num_segments=n)

    deg_s = seg(ew, src) + 1.0
    deg_d = seg(ew, dst) + 1.0
    norm = ew * jax.lax.rsqrt(deg_s)[src] * jax.lax.rsqrt(deg_d)[dst]
    es = x @ (W_gat @ a_src)
    ed = x @ (W_gat @ a_dst)
    ee = jnp.exp(jax.nn.leaky_relu(es[src] + ed[dst], 0.2))
    den = seg(ee, dst)
    invden = jnp.where(den > 0, 1.0 / den, 0.0)
    coef = ee * invden[dst]
    cnt = seg(jnp.ones_like(ew), dst)
    invc = (1.0 / jnp.maximum(cnt, 1.0)).reshape(n, 1)

    def cmat(w):
        c = jnp.zeros((n, 16), dtype=jnp.float32)
        for k in range(3):
            c = c.at[dst, eidx[:, k]].add(w)
        return c

    ccat = jnp.concatenate(
        [cmat(norm), cmat(jnp.ones_like(ew)), cmat(ew), cmat(coef)], axis=1)

    h = x @ W_gat
    xs = x[src]
    bond_gin = be_gin[eidx].sum(axis=1)
    s1 = seg(norm[:, None] * xs, dst)
    s2 = seg(jax.nn.relu(xs + bond_gin), dst)
    s3 = seg(coef[:, None] * h[src], dst)
    s4 = seg(xs, dst)
    s5 = seg(ew[:, None] * xs, dst)

    pad = jnp.zeros((1, D), jnp.float32)
    padded = lambda m: jnp.concatenate([m, pad], axis=0)
    z16 = jnp.zeros((16, D), jnp.float32)
    pgcn = jnp.concatenate([padded(be_gcn @ W_gcn), z16, z16, z16], axis=0)
    pgat = jnp.concatenate([z16, z16, z16, padded(be_gat)], axis=0)
    psage = jnp.concatenate([z16, padded(be_sage @ W_sage_nbr), z16, z16], axis=0)
    pgraph = jnp.concatenate([z16, z16, padded(be_graph @ W_graph2), z16], axis=0)
    parma = jnp.concatenate([padded(be_arma @ W_arma1), z16, z16, z16], axis=0)
    pc = jnp.concatenate([pgcn, pgat, psage, pgraph, parma], axis=1)
    wx = jnp.concatenate(
        [(1.0 + eps_gin) * W_gin1, W_sage_self, W_graph1, W_arma2, W_mlp1],
        axis=1)
    ws1 = jnp.concatenate([W_gcn, W_arma1], axis=1)
    jmap = jnp.array([[(o * 256 + b * 32) % 224 // 32 for b in range(8)]
                      for o in range(7)], dtype=jnp.int32)
    wmix = jnp.repeat(weights[jmap], CS, axis=1)

    _ = chunk_eye
    return _dense_stage(x, s1, s2, s3, s4, s5, ccat, invc, wx, ws1, W_gin1,
                        W_sage_nbr, W_graph2, W_gin2, W_mlp2, pc,
                        ln_gamma.reshape(1, D), ln_beta.reshape(1, D), wmix)
